# baseline (speedup 1.0000x reference)
import functools

import jax
import jax.numpy as jnp
from jax import lax
from jax.experimental import pallas as pl
from jax.experimental.pallas import tpu as pltpu
from jax.experimental.pallas import tpu_sc as plsc

_COEFF = 0.01
_BIG_I = 2**30
_BLK = 256
_HT = 256
_NC = 2
_NS = 16
_E = 8


def _router_body(x_ref, wg_ref, ea_ref, wa_ref, aux_ref):
    xf = x_ref[...]
    wg = wg_ref[...]
    T, _ = xf.shape
    E = wg.shape[0]
    logits = jnp.dot(xf, wg.T, preferred_element_type=jnp.float32)
    iota = lax.broadcasted_iota(jnp.int32, logits.shape, 1)
    m1 = jnp.max(logits, axis=-1, keepdims=True)
    a1 = jnp.min(jnp.where(logits == m1, iota, _BIG_I), axis=-1, keepdims=True)
    oh1 = (iota == a1)
    rest = jnp.where(oh1, -jnp.inf, logits)
    m2 = jnp.max(rest, axis=-1, keepdims=True)
    a2 = jnp.min(jnp.where(rest == m2, iota, _BIG_I), axis=-1, keepdims=True)
    e2 = jnp.exp(m2 - m1)
    denom = 1.0 + e2
    ea_ref[...] = jnp.concatenate([a1, a2], axis=1)
    wa_ref[...] = jnp.concatenate([1.0 / denom, e2 / denom], axis=1)
    gates = jnp.exp(logits - m1)
    gates = gates / jnp.sum(gates, axis=-1, keepdims=True)
    f = jnp.mean(oh1.astype(jnp.float32), axis=0)
    P = jnp.mean(gates, axis=0)
    aux_ref[0, 0] = _COEFF * E * jnp.sum(f * P)


def _gelu(g):
    return 0.5 * g * (1.0 + lax.erf(g * (2.0 ** -0.5)))


def _moe_ffn_body(blk_e_ref, nblk_ref, xg_ref, gw_ref, uw_ref, dw_ref, y_ref):
    b = pl.program_id(0)
    h = pl.program_id(1)

    @pl.when(b < nblk_ref[0])
    def _():
        xblk = xg_ref[...].astype(jnp.bfloat16)
        g = jnp.dot(xblk, gw_ref[0].T, preferred_element_type=jnp.float32)
        u = jnp.dot(xblk, uw_ref[0].T, preferred_element_type=jnp.float32)
        hpart = (_gelu(g) * u).astype(jnp.bfloat16)

        @pl.when(h == 0)
        def _():
            y_ref[...] = jnp.zeros_like(y_ref)

        y_ref[...] += jnp.dot(hpart, dw_ref[0], preferred_element_type=jnp.float32)


def _sh_ffn_body(x_ref, gw_ref, uw_ref, dw_ref, y_ref):
    h = pl.program_id(1)
    xblk = x_ref[...].astype(jnp.bfloat16)
    g = jnp.dot(xblk, gw_ref[...].T, preferred_element_type=jnp.float32)
    u = jnp.dot(xblk, uw_ref[...].T, preferred_element_type=jnp.float32)
    hpart = (_gelu(g) * u).astype(jnp.bfloat16)

    @pl.when(h == 0)
    def _():
        y_ref[...] = jnp.zeros_like(y_ref)

    y_ref[...] += jnp.dot(hpart, dw_ref[...], preferred_element_type=jnp.float32)


def _iota16():
    return lax.broadcasted_iota(jnp.int32, (16,), 0)


def _lane(vec, e):
    return jnp.sum(jnp.where(_iota16() == e, vec, 0))


def _dispatch_body(NA, P, ea_hbm, pos_hbm, src_hbm, blk_hbm, nblk_hbm,
                   ea_v, ea_own_v, loc_v, cnt_v, pidx_v, val_v,
                   padi_v, padz_v, blk_v, nblk_v, shared_cnt, sem):
    c = lax.axis_index("c")
    s = lax.axis_index("s")
    oc = 2 * s + c
    iota = _iota16()
    CA = NA // (_NC * _NS)
    NV = CA // 16

    pltpu.sync_copy(ea_hbm.at[pl.ds(s * 2 * CA, 2 * CA)], ea_v)
    for cl in range(2):
        cnt = [jnp.int32(0)] * _E
        for v in range(NV):
            vec = ea_v[pl.ds(cl * CA + v * 16, 16)]
            for e in range(_E):
                cnt[e] = cnt[e] + jnp.sum((vec == e).astype(jnp.int32))
        row = jnp.zeros((16,), jnp.int32)
        for e in range(_E):
            row = jnp.where(iota == e, cnt[e], row)
        loc_v[pl.ds(cl * 16, 16)] = row
    pltpu.sync_copy(loc_v, shared_cnt.at[pl.ds(s * 32, 32)])
    plsc.subcore_barrier()

    pltpu.sync_copy(shared_cnt, cnt_v)
    totals = jnp.zeros((16,), jnp.int32)
    prior = jnp.zeros((16,), jnp.int32)
    for w in range(_NC * _NS):
        roww = cnt_v[pl.ds(w * 16, 16)]
        totals = totals + roww
        prior = prior + roww * jnp.where(jnp.int32(w) < oc, 1, 0)
    padded = ((totals + (_BLK - 1)) >> 8) << 8
    inc_pad = plsc.cumsum(padded)
    pstart = inc_pad - padded
    pend = inc_pad
    base_vec = pstart + prior

    pltpu.sync_copy(ea_hbm.at[pl.ds(oc * CA, CA)], ea_own_v)
    base = [_lane(base_vec, e) for e in range(_E)]
    for v in range(NV):
        vec = ea_own_v[pl.ds(v * 16, 16)]
        poss = jnp.zeros((16,), jnp.int32)
        for e in range(_E):
            m = vec == e
            mi = m.astype(jnp.int32)
            incl = plsc.cumsum(mi)
            poss = jnp.where(m, base[e] + incl - 1, poss)
            base[e] = base[e] + jnp.sum(mi)
        pidx_v[pl.ds(v * 16, 16)] = poss
        j = oc * CA + v * 16 + iota
        val_v[pl.ds(v * 16, 16)] = j >> 1
    pltpu.sync_copy(pidx_v, pos_hbm.at[pl.ds(oc * CA, CA)])
    pltpu.async_copy(val_v, src_hbm.at[pidx_v], sem).wait()

    @pl.when(oc < _E)
    def _():
        pst = _lane(pstart, oc) + _lane(totals, oc)
        pen = _lane(pend, oc)

        @pl.when(pen > pst)
        def _():
            for half in range(2):
                for u in range(8):
                    idxv = jnp.minimum(pst + (half * 128 + u * 16) + iota,
                                       pen - 1)
                    padi_v[pl.ds(u * 16, 16)] = idxv
                    padz_v[pl.ds(u * 16, 16)] = jnp.zeros((16,), jnp.int32)
                pltpu.async_copy(padz_v, src_hbm.at[padi_v], sem).wait()

    @pl.when(oc == 0)
    def _():
        pens = [_lane(pend, e) for e in range(_E)]
        for u in range(2):
            bs = (iota + u * 16) * _BLK
            cntv = jnp.zeros((16,), jnp.int32)
            for e in range(_E):
                cntv = cntv + (bs >= pens[e]).astype(jnp.int32)
            blk_v[pl.ds(u * 16, 16)] = jnp.minimum(cntv, _E - 1)
        pltpu.sync_copy(blk_v, blk_hbm)
        nblk = jnp.sum(padded) >> 8
        nblk_v[...] = jnp.where(iota == 0, nblk, 0)
        pltpu.sync_copy(nblk_v, nblk_hbm)


def _xgather_body(P, D, src_hbm, x_hbm, xg_hbm, idx_v, rows_v, sem):
    c = lax.axis_index("c")
    s = lax.axis_index("s")
    g = 2 * s + c
    RPW = P // (_NC * _NS)
    NCH = RPW // 64
    for cc in range(NCH):
        base = g * RPW + cc * 64
        pltpu.sync_copy(src_hbm.at[pl.ds(base, 64)], idx_v)
        for u in range(4):
            iv = idx_v[pl.ds(u * 16, 16)]
            idx_v[pl.ds(u * 16, 16)] = jnp.clip(iv, 0, x_hbm.shape[0] - 1)
        pltpu.async_copy(x_hbm.at[idx_v], rows_v, sem).wait()
        pltpu.sync_copy(rows_v, xg_hbm.at[pl.ds(base, 64)])


def _combine_body(TT, D, y_hbm, ysh_hbm, pos_hbm, wa_hbm, out_hbm,
                  pos_v, wa_v, rows_v, ysh_v, out_v, sem):
    c = lax.axis_index("c")
    s = lax.axis_index("s")
    g = 2 * s + c
    TPW = TT // (_NC * _NS)
    pltpu.sync_copy(wa_hbm.at[pl.ds(g * 2 * TPW, 2 * TPW)], wa_v)
    NM = D // 16
    for cc in range(TPW // 16):
        tb = g * TPW + cc * 16
        pltpu.sync_copy(pos_hbm.at[pl.ds(tb * 2, 32)], pos_v)
        cp = pltpu.async_copy(y_hbm.at[pos_v], rows_v, sem)
        pltpu.sync_copy(ysh_hbm.at[pl.ds(tb, 16)], ysh_v)
        cp.wait()
        wv_a = wa_v[pl.ds(cc * 32, 16)]
        wv_b = wa_v[pl.ds(cc * 32 + 16, 16)]
        for t in range(16):
            wv = wv_a if t < 8 else wv_b
            w0 = wv[(2 * t) % 16]
            w1 = wv[(2 * t + 1) % 16]

            def mbody(m, _, t=t, w0=w0, w1=w1):
                sl = pl.ds(m * 16, 16)
                out_v[t, sl] = (w0 * rows_v[2 * t, sl]
                                + w1 * rows_v[2 * t + 1, sl]
                                + ysh_v[t, sl])
                return 0

            lax.fori_loop(0, NM, mbody, 0)
        pltpu.sync_copy(out_v, out_hbm.at[pl.ds(tb, 16)])


def kernel(x, Wg, gate_w, up_w, down_w, sh_gate, sh_up, sh_down):
    B, T, D = x.shape
    E, H, _ = gate_w.shape
    TT = B * T
    NA = 2 * TT
    NBLK = (NA + E * (_BLK - 1)) // _BLK + 1
    P = NBLK * _BLK
    NH = H // _HT
    xf = x.reshape(TT, D)

    ea, wa, aux = pl.pallas_call(
        _router_body,
        out_shape=(
            jax.ShapeDtypeStruct((TT, 2), jnp.int32),
            jax.ShapeDtypeStruct((TT, 2), jnp.float32),
            jax.ShapeDtypeStruct((1, 1), jnp.float32),
        ),
        out_specs=(
            pl.BlockSpec(memory_space=pltpu.VMEM),
            pl.BlockSpec(memory_space=pltpu.VMEM),
            pl.BlockSpec(memory_space=pltpu.SMEM),
        ),
    )(xf, Wg)
    eaf = ea.reshape(NA)
    waf = wa.reshape(NA)

    mesh = plsc.VectorSubcoreMesh(core_axis_name="c", subcore_axis_name="s",
                                  num_cores=_NC, num_subcores=_NS)

    d1 = pl.kernel(
        functools.partial(_dispatch_body, NA, P),
        compiler_params=pltpu.CompilerParams(needs_layout_passes=False),
        out_type=(
            jax.ShapeDtypeStruct((NA,), jnp.int32),
            jax.ShapeDtypeStruct((P,), jnp.int32),
            jax.ShapeDtypeStruct((32,), jnp.int32),
            jax.ShapeDtypeStruct((16,), jnp.int32),
        ),
        mesh=mesh,
        scratch_types=[
            pltpu.VMEM((256,), jnp.int32),
            pltpu.VMEM((128,), jnp.int32),
            pltpu.VMEM((32,), jnp.int32),
            pltpu.VMEM((512,), jnp.int32),
            pltpu.VMEM((128,), jnp.int32),
            pltpu.VMEM((128,), jnp.int32),
            pltpu.VMEM((128,), jnp.int32),
            pltpu.VMEM((128,), jnp.int32),
            pltpu.VMEM((32,), jnp.int32),
            pltpu.VMEM((16,), jnp.int32),
            pltpu.VMEM_SHARED((512,), jnp.int32),
            pltpu.SemaphoreType.DMA,
        ],
    )
    pos, src, blk_e, nblk = d1(eaf)

    d2 = pl.kernel(
        functools.partial(_xgather_body, P, D),
        out_type=jax.ShapeDtypeStruct((P, D), jnp.float32),
        mesh=mesh,
        scratch_types=[
            pltpu.VMEM((64,), jnp.int32),
            pltpu.VMEM((64, D), jnp.float32),
            pltpu.SemaphoreType.DMA,
        ],
    )
    xg = d2(src, xf)

    gw16 = gate_w.astype(jnp.bfloat16)
    uw16 = up_w.astype(jnp.bfloat16)
    dw16 = jnp.swapaxes(down_w, 1, 2).astype(jnp.bfloat16)

    grid_spec = pltpu.PrefetchScalarGridSpec(
        num_scalar_prefetch=2,
        grid=(NBLK, NH),
        in_specs=[
            pl.BlockSpec((_BLK, D), lambda b, h, be, nb: (b, 0)),
            pl.BlockSpec((1, _HT, D), lambda b, h, be, nb: (be[b], h, 0)),
            pl.BlockSpec((1, _HT, D), lambda b, h, be, nb: (be[b], h, 0)),
            pl.BlockSpec((1, _HT, D), lambda b, h, be, nb: (be[b], h, 0)),
        ],
        out_specs=pl.BlockSpec((_BLK, D), lambda b, h, be, nb: (b, 0)),
    )
    y = pl.pallas_call(
        _moe_ffn_body,
        grid_spec=grid_spec,
        out_shape=jax.ShapeDtypeStruct((P, D), jnp.float32),
        compiler_params=pltpu.CompilerParams(
            dimension_semantics=("arbitrary", "arbitrary"),
        ),
    )(blk_e, nblk, xg, gw16, uw16, dw16)

    shg16 = sh_gate.astype(jnp.bfloat16)
    shu16 = sh_up.astype(jnp.bfloat16)
    shd16 = jnp.swapaxes(sh_down, 0, 1).astype(jnp.bfloat16)
    TB = min(512, TT)
    NTB = TT // TB
    ysh = pl.pallas_call(
        _sh_ffn_body,
        grid=(NTB, NH),
        in_specs=[
            pl.BlockSpec((TB, D), lambda b, h: (b, 0)),
            pl.BlockSpec((_HT, D), lambda b, h: (h, 0)),
            pl.BlockSpec((_HT, D), lambda b, h: (h, 0)),
            pl.BlockSpec((_HT, D), lambda b, h: (h, 0)),
        ],
        out_specs=pl.BlockSpec((TB, D), lambda b, h: (b, 0)),
        out_shape=jax.ShapeDtypeStruct((TT, D), jnp.float32),
        compiler_params=pltpu.CompilerParams(
            dimension_semantics=("arbitrary", "arbitrary"),
        ),
    )(xf, shg16, shu16, shd16)

    d3 = pl.kernel(
        functools.partial(_combine_body, TT, D),
        out_type=jax.ShapeDtypeStruct((TT, D), jnp.float32),
        mesh=mesh,
        scratch_types=[
            pltpu.VMEM((32,), jnp.int32),
            pltpu.VMEM((128,), jnp.float32),
            pltpu.VMEM((32, D), jnp.float32),
            pltpu.VMEM((16, D), jnp.float32),
            pltpu.VMEM((16, D), jnp.float32),
            pltpu.SemaphoreType.DMA,
        ],
    )
    out = d3(y, ysh, pos, waf)

    return out.reshape(B, T, D), aux.reshape(())


# speedup vs baseline: 1.2433x; 1.2433x over previous
import functools

import jax
import jax.numpy as jnp
from jax import lax
from jax.experimental import pallas as pl
from jax.experimental.pallas import tpu as pltpu
from jax.experimental.pallas import tpu_sc as plsc

_COEFF = 0.01
_BIG_I = 2**30
_BLK = 256
_HT = 256
_NC = 2
_NS = 16
_E = 8


def _router_body(x_ref, wg_ref, ea_ref, wa_ref, aux_ref):
    xf = x_ref[...]
    wg = wg_ref[...]
    T, _ = xf.shape
    E = wg.shape[0]
    logits = jnp.dot(xf, wg.T, preferred_element_type=jnp.float32)
    iota = lax.broadcasted_iota(jnp.int32, logits.shape, 1)
    m1 = jnp.max(logits, axis=-1, keepdims=True)
    a1 = jnp.min(jnp.where(logits == m1, iota, _BIG_I), axis=-1, keepdims=True)
    oh1 = (iota == a1)
    rest = jnp.where(oh1, -jnp.inf, logits)
    m2 = jnp.max(rest, axis=-1, keepdims=True)
    a2 = jnp.min(jnp.where(rest == m2, iota, _BIG_I), axis=-1, keepdims=True)
    e2 = jnp.exp(m2 - m1)
    denom = 1.0 + e2
    ea_ref[...] = jnp.concatenate([a1, a2], axis=1)
    wa_ref[...] = jnp.concatenate([1.0 / denom, e2 / denom], axis=1)
    gates = jnp.exp(logits - m1)
    gates = gates / jnp.sum(gates, axis=-1, keepdims=True)
    f = jnp.mean(oh1.astype(jnp.float32), axis=0)
    P = jnp.mean(gates, axis=0)
    aux_ref[0, 0] = _COEFF * E * jnp.sum(f * P)


def _gelu(g):
    return 0.5 * g * (1.0 + lax.erf(g * (2.0 ** -0.5)))


def _moe_ffn_body(blk_e_ref, nblk_ref, xg_ref, gw_ref, uw_ref, dw_ref, y_ref):
    b = pl.program_id(0)
    h = pl.program_id(1)

    @pl.when(b < nblk_ref[0])
    def _():
        xblk = xg_ref[...].astype(jnp.bfloat16)
        g = jnp.dot(xblk, gw_ref[0].T, preferred_element_type=jnp.float32)
        u = jnp.dot(xblk, uw_ref[0].T, preferred_element_type=jnp.float32)
        hpart = (_gelu(g) * u).astype(jnp.bfloat16)

        @pl.when(h == 0)
        def _():
            y_ref[...] = jnp.zeros_like(y_ref)

        y_ref[...] += jnp.dot(hpart, dw_ref[0], preferred_element_type=jnp.float32)


def _sh_ffn_body(x_ref, gw_ref, uw_ref, dw_ref, y_ref):
    h = pl.program_id(1)
    xblk = x_ref[...].astype(jnp.bfloat16)
    g = jnp.dot(xblk, gw_ref[...].T, preferred_element_type=jnp.float32)
    u = jnp.dot(xblk, uw_ref[...].T, preferred_element_type=jnp.float32)
    hpart = (_gelu(g) * u).astype(jnp.bfloat16)

    @pl.when(h == 0)
    def _():
        y_ref[...] = jnp.zeros_like(y_ref)

    y_ref[...] += jnp.dot(hpart, dw_ref[...], preferred_element_type=jnp.float32)


def _iota16():
    return lax.broadcasted_iota(jnp.int32, (16,), 0)


def _lane(vec, e):
    return jnp.sum(jnp.where(_iota16() == e, vec, 0))


def _dispatch_body(NA, P, ea_hbm, x_hbm, pos_hbm, xg_hbm, blk_hbm, nblk_hbm,
                   ea_v, loc_v, cnt_v, pstage_v, pidx_v, tok_v, blk_v, nblk_v,
                   rows_a, rows_b, shared_cnt, gsem, ssem):
    c = lax.axis_index("c")
    s = lax.axis_index("s")
    oc = 2 * s + c
    iota = _iota16()
    CA = NA // (_NC * _NS)
    NV = CA // 16

    pltpu.sync_copy(ea_hbm.at[pl.ds(s * 2 * CA, 2 * CA)], ea_v)
    for cl in range(2):
        cnt = [jnp.int32(0)] * _E
        for v in range(NV):
            vec = ea_v[pl.ds(cl * CA + v * 16, 16)]
            for e in range(_E):
                cnt[e] = cnt[e] + jnp.sum((vec == e).astype(jnp.int32))
        row = jnp.zeros((16,), jnp.int32)
        for e in range(_E):
            row = jnp.where(iota == e, cnt[e], row)
        loc_v[pl.ds(cl * 16, 16)] = row
    pltpu.sync_copy(loc_v, shared_cnt.at[pl.ds(s * 32, 32)])
    plsc.subcore_barrier()

    pltpu.sync_copy(shared_cnt, cnt_v)
    totals = jnp.zeros((16,), jnp.int32)
    prior = jnp.zeros((16,), jnp.int32)
    for w in range(_NC * _NS):
        roww = cnt_v[pl.ds(w * 16, 16)]
        totals = totals + roww
        prior = prior + roww * jnp.where(jnp.int32(w) < oc, 1, 0)
    padded = ((totals + (_BLK - 1)) >> 8) << 8
    inc_pad = plsc.cumsum(padded)
    pstart = inc_pad - padded
    pend = inc_pad
    base_vec = pstart + prior

    base = [_lane(base_vec, e) for e in range(_E)]
    for v in range(NV):
        va = ea_v[pl.ds(v * 16, 16)]
        vb = ea_v[pl.ds(CA + v * 16, 16)]
        vec = jnp.where(c == 1, vb, va)
        poss = jnp.zeros((16,), jnp.int32)
        for e in range(_E):
            m = vec == e
            mi = m.astype(jnp.int32)
            incl = plsc.cumsum(mi)
            poss = jnp.where(m, base[e] + incl - 1, poss)
            base[e] = base[e] + jnp.sum(mi)
        pstage_v[pl.ds(v * 16, 16)] = poss
        pidx_v[v // 2][pl.ds((v % 2) * 16, 16)] = poss
        j = oc * CA + v * 16 + iota
        tok_v[v // 2][pl.ds((v % 2) * 16, 16)] = j >> 1
    pltpu.sync_copy(pstage_v, pos_hbm.at[pl.ds(oc * CA, CA)])

    @pl.when(oc == 0)
    def _():
        pens = [_lane(pend, e) for e in range(_E)]
        for u in range(2):
            bs = (iota + u * 16) * _BLK
            cntv = jnp.zeros((16,), jnp.int32)
            for e in range(_E):
                cntv = cntv + (bs >= pens[e]).astype(jnp.int32)
            blk_v[pl.ds(u * 16, 16)] = jnp.minimum(cntv, _E - 1)
        pltpu.sync_copy(blk_v, blk_hbm)
        nblk = jnp.sum(padded) >> 8
        nblk_v[...] = jnp.where(iota == 0, nblk, 0)
        pltpu.sync_copy(nblk_v, nblk_hbm)

    bufs = (rows_a, rows_b)
    gh = [None] * 4
    sh = [None] * 4
    gh[0] = pltpu.async_copy(x_hbm.at[tok_v[0]], bufs[0], gsem)
    for cc in range(4):
        gh[cc].wait()
        if cc + 1 < 4:
            if cc >= 1:
                sh[cc - 1].wait()
            gh[cc + 1] = pltpu.async_copy(x_hbm.at[tok_v[cc + 1]],
                                          bufs[(cc + 1) % 2], gsem)
        sh[cc] = pltpu.async_copy(bufs[cc % 2], xg_hbm.at[pidx_v[cc]], ssem)
    sh[2].wait()
    sh[3].wait()


def _combine_body(TT, D, y_hbm, ysh_hbm, pos_hbm, wa_hbm, out_hbm,
                  pos_v, wa_v, rows_v, ysh_v, out_v, sem):
    c = lax.axis_index("c")
    s = lax.axis_index("s")
    g = 2 * s + c
    TPW = TT // (_NC * _NS)
    pltpu.sync_copy(wa_hbm.at[pl.ds(g * 2 * TPW, 2 * TPW)], wa_v)
    NM = D // 16
    for cc in range(TPW // 16):
        tb = g * TPW + cc * 16
        pltpu.sync_copy(pos_hbm.at[pl.ds(tb * 2, 32)], pos_v)
        cp = pltpu.async_copy(y_hbm.at[pos_v], rows_v, sem)
        pltpu.sync_copy(ysh_hbm.at[pl.ds(tb, 16)], ysh_v)
        cp.wait()
        wv_a = wa_v[pl.ds(cc * 32, 16)]
        wv_b = wa_v[pl.ds(cc * 32 + 16, 16)]
        for t in range(16):
            wv = wv_a if t < 8 else wv_b
            w0 = wv[(2 * t) % 16]
            w1 = wv[(2 * t + 1) % 16]

            def mbody(m, _, t=t, w0=w0, w1=w1):
                sl = pl.ds(m * 16, 16)
                out_v[t, sl] = (w0 * rows_v[2 * t, sl]
                                + w1 * rows_v[2 * t + 1, sl]
                                + ysh_v[t, sl])
                return 0

            lax.fori_loop(0, NM, mbody, 0)
        pltpu.sync_copy(out_v, out_hbm.at[pl.ds(tb, 16)])


def kernel(x, Wg, gate_w, up_w, down_w, sh_gate, sh_up, sh_down):
    B, T, D = x.shape
    E, H, _ = gate_w.shape
    TT = B * T
    NA = 2 * TT
    NBLK = (NA + E * (_BLK - 1)) // _BLK + 1
    P = NBLK * _BLK
    NH = H // _HT
    xf = x.reshape(TT, D)

    ea, wa, aux = pl.pallas_call(
        _router_body,
        out_shape=(
            jax.ShapeDtypeStruct((TT, 2), jnp.int32),
            jax.ShapeDtypeStruct((TT, 2), jnp.float32),
            jax.ShapeDtypeStruct((1, 1), jnp.float32),
        ),
        out_specs=(
            pl.BlockSpec(memory_space=pltpu.VMEM),
            pl.BlockSpec(memory_space=pltpu.VMEM),
            pl.BlockSpec(memory_space=pltpu.SMEM),
        ),
    )(xf, Wg)
    eaf = ea.reshape(NA)
    waf = wa.reshape(NA)

    mesh = plsc.VectorSubcoreMesh(core_axis_name="c", subcore_axis_name="s",
                                  num_cores=_NC, num_subcores=_NS)

    d1 = pl.kernel(
        functools.partial(_dispatch_body, NA, P),
        compiler_params=pltpu.CompilerParams(needs_layout_passes=False),
        out_type=(
            jax.ShapeDtypeStruct((NA,), jnp.int32),
            jax.ShapeDtypeStruct((P, D), jnp.float32),
            jax.ShapeDtypeStruct((32,), jnp.int32),
            jax.ShapeDtypeStruct((16,), jnp.int32),
        ),
        mesh=mesh,
        scratch_types=[
            pltpu.VMEM((256,), jnp.int32),
            pltpu.VMEM((32,), jnp.int32),
            pltpu.VMEM((512,), jnp.int32),
            pltpu.VMEM((128,), jnp.int32),
            tuple(pltpu.VMEM((32,), jnp.int32) for _ in range(4)),
            tuple(pltpu.VMEM((32,), jnp.int32) for _ in range(4)),
            pltpu.VMEM((32,), jnp.int32),
            pltpu.VMEM((16,), jnp.int32),
            pltpu.VMEM((32, D), jnp.float32),
            pltpu.VMEM((32, D), jnp.float32),
            pltpu.VMEM_SHARED((512,), jnp.int32),
            pltpu.SemaphoreType.DMA,
            pltpu.SemaphoreType.DMA,
        ],
    )
    pos, xg, blk_e, nblk = d1(eaf, xf)

    gw16 = gate_w.astype(jnp.bfloat16)
    uw16 = up_w.astype(jnp.bfloat16)
    dw16 = jnp.swapaxes(down_w, 1, 2).astype(jnp.bfloat16)

    grid_spec = pltpu.PrefetchScalarGridSpec(
        num_scalar_prefetch=2,
        grid=(NBLK, NH),
        in_specs=[
            pl.BlockSpec((_BLK, D), lambda b, h, be, nb: (b, 0)),
            pl.BlockSpec((1, _HT, D), lambda b, h, be, nb: (be[b], h, 0)),
            pl.BlockSpec((1, _HT, D), lambda b, h, be, nb: (be[b], h, 0)),
            pl.BlockSpec((1, _HT, D), lambda b, h, be, nb: (be[b], h, 0)),
        ],
        out_specs=pl.BlockSpec((_BLK, D), lambda b, h, be, nb: (b, 0)),
    )
    y = pl.pallas_call(
        _moe_ffn_body,
        grid_spec=grid_spec,
        out_shape=jax.ShapeDtypeStruct((P, D), jnp.float32),
        compiler_params=pltpu.CompilerParams(
            dimension_semantics=("arbitrary", "arbitrary"),
        ),
    )(blk_e, nblk, xg, gw16, uw16, dw16)

    shg16 = sh_gate.astype(jnp.bfloat16)
    shu16 = sh_up.astype(jnp.bfloat16)
    shd16 = jnp.swapaxes(sh_down, 0, 1).astype(jnp.bfloat16)
    TB = min(512, TT)
    NTB = TT // TB
    ysh = pl.pallas_call(
        _sh_ffn_body,
        grid=(NTB, NH),
        in_specs=[
            pl.BlockSpec((TB, D), lambda b, h: (b, 0)),
            pl.BlockSpec((_HT, D), lambda b, h: (h, 0)),
            pl.BlockSpec((_HT, D), lambda b, h: (h, 0)),
            pl.BlockSpec((_HT, D), lambda b, h: (h, 0)),
        ],
        out_specs=pl.BlockSpec((TB, D), lambda b, h: (b, 0)),
        out_shape=jax.ShapeDtypeStruct((TT, D), jnp.float32),
        compiler_params=pltpu.CompilerParams(
            dimension_semantics=("arbitrary", "arbitrary"),
        ),
    )(xf, shg16, shu16, shd16)

    d3 = pl.kernel(
        functools.partial(_combine_body, TT, D),
        out_type=jax.ShapeDtypeStruct((TT, D), jnp.float32),
        mesh=mesh,
        scratch_types=[
            pltpu.VMEM((32,), jnp.int32),
            pltpu.VMEM((128,), jnp.float32),
            pltpu.VMEM((32, D), jnp.float32),
            pltpu.VMEM((16, D), jnp.float32),
            pltpu.VMEM((16, D), jnp.float32),
            pltpu.SemaphoreType.DMA,
        ],
    )
    out = d3(y, ysh, pos, waf)

    return out.reshape(B, T, D), aux.reshape(())


# speedup vs baseline: 1.4736x; 1.1852x over previous
import functools

import jax
import jax.numpy as jnp
from jax import lax
from jax.experimental import pallas as pl
from jax.experimental.pallas import tpu as pltpu
from jax.experimental.pallas import tpu_sc as plsc

_COEFF = 0.01
_BIG_I = 2**30
_BLK = 256
_HT = 256
_NC = 2
_NS = 16
_E = 8


def _router_body(x_ref, wg_ref, ea_ref, wa_ref, aux_ref):
    xf = x_ref[...]
    wg = wg_ref[...]
    T, _ = xf.shape
    E = wg.shape[0]
    logits = jnp.dot(xf, wg.T, preferred_element_type=jnp.float32)
    iota = lax.broadcasted_iota(jnp.int32, logits.shape, 1)
    m1 = jnp.max(logits, axis=-1, keepdims=True)
    a1 = jnp.min(jnp.where(logits == m1, iota, _BIG_I), axis=-1, keepdims=True)
    oh1 = (iota == a1)
    rest = jnp.where(oh1, -jnp.inf, logits)
    m2 = jnp.max(rest, axis=-1, keepdims=True)
    a2 = jnp.min(jnp.where(rest == m2, iota, _BIG_I), axis=-1, keepdims=True)
    e2 = jnp.exp(m2 - m1)
    denom = 1.0 + e2
    ea_ref[...] = jnp.concatenate([a1, a2], axis=1)
    wa_ref[...] = jnp.concatenate([1.0 / denom, e2 / denom], axis=1)
    gates = jnp.exp(logits - m1)
    gates = gates / jnp.sum(gates, axis=-1, keepdims=True)
    f = jnp.mean(oh1.astype(jnp.float32), axis=0)
    P = jnp.mean(gates, axis=0)
    aux_ref[0, 0] = _COEFF * E * jnp.sum(f * P)


def _gelu(g):
    return 0.5 * g * (1.0 + lax.erf(g * (2.0 ** -0.5)))


def _moe_ffn_body(blk_e_ref, nblk_ref, xg_ref, gw_ref, uw_ref, dw_ref, y_ref):
    b = pl.program_id(0)
    h = pl.program_id(1)

    @pl.when(b < nblk_ref[0])
    def _():
        xblk = xg_ref[...]
        g = jnp.dot(xblk, gw_ref[0].T, preferred_element_type=jnp.float32)
        u = jnp.dot(xblk, uw_ref[0].T, preferred_element_type=jnp.float32)
        hpart = _gelu(g) * u
        yp = lax.dot_general(hpart, dw_ref[0], (((1,), (1,)), ((), ())),
                             preferred_element_type=jnp.float32)

        @pl.when(h == 0)
        def _():
            y_ref[...] = jnp.zeros_like(y_ref)

        y_ref[...] += yp


def _sh_ffn_body(x_ref, gw_ref, uw_ref, dw_ref, y_ref):
    h = pl.program_id(1)
    xblk = x_ref[...]
    g = jnp.dot(xblk, gw_ref[...].T, preferred_element_type=jnp.float32)
    u = jnp.dot(xblk, uw_ref[...].T, preferred_element_type=jnp.float32)
    hpart = _gelu(g) * u
    yp = lax.dot_general(hpart, dw_ref[...], (((1,), (1,)), ((), ())),
                         preferred_element_type=jnp.float32)

    @pl.when(h == 0)
    def _():
        y_ref[...] = jnp.zeros_like(y_ref)

    y_ref[...] += yp


def _iota16():
    return lax.broadcasted_iota(jnp.int32, (16,), 0)


def _lane(vec, e):
    return jnp.sum(jnp.where(_iota16() == e, vec, 0))


def _dispatch_body(NA, P, ea_hbm, x_hbm, pos_hbm, xg_hbm, blk_hbm, nblk_hbm,
                   ea_v, loc_v, cnt_v, pstage_v, pidx_v, tok_v, blk_v, nblk_v,
                   rows_a, rows_b, shared_cnt, gsem, ssem):
    c = lax.axis_index("c")
    s = lax.axis_index("s")
    oc = 2 * s + c
    iota = _iota16()
    CA = NA // (_NC * _NS)
    NV = CA // 16

    pltpu.sync_copy(ea_hbm.at[pl.ds(s * 2 * CA, 2 * CA)], ea_v)
    for cl in range(2):
        cnt = [jnp.int32(0)] * _E
        for v in range(NV):
            vec = ea_v[pl.ds(cl * CA + v * 16, 16)]
            for e in range(_E):
                cnt[e] = cnt[e] + jnp.sum((vec == e).astype(jnp.int32))
        row = jnp.zeros((16,), jnp.int32)
        for e in range(_E):
            row = jnp.where(iota == e, cnt[e], row)
        loc_v[pl.ds(cl * 16, 16)] = row
    pltpu.sync_copy(loc_v, shared_cnt.at[pl.ds(s * 32, 32)])
    plsc.subcore_barrier()

    pltpu.sync_copy(shared_cnt, cnt_v)
    totals = jnp.zeros((16,), jnp.int32)
    prior = jnp.zeros((16,), jnp.int32)
    for w in range(_NC * _NS):
        roww = cnt_v[pl.ds(w * 16, 16)]
        totals = totals + roww
        prior = prior + roww * jnp.where(jnp.int32(w) < oc, 1, 0)
    padded = ((totals + (_BLK - 1)) >> 8) << 8
    inc_pad = plsc.cumsum(padded)
    pstart = inc_pad - padded
    pend = inc_pad
    base_vec = pstart + prior

    base = [_lane(base_vec, e) for e in range(_E)]
    for v in range(NV):
        va = ea_v[pl.ds(v * 16, 16)]
        vb = ea_v[pl.ds(CA + v * 16, 16)]
        vec = jnp.where(c == 1, vb, va)
        poss = jnp.zeros((16,), jnp.int32)
        for e in range(_E):
            m = vec == e
            mi = m.astype(jnp.int32)
            incl = plsc.cumsum(mi)
            poss = jnp.where(m, base[e] + incl - 1, poss)
            base[e] = base[e] + jnp.sum(mi)
        pstage_v[pl.ds(v * 16, 16)] = poss
        pidx_v[v // 2][pl.ds((v % 2) * 16, 16)] = poss
        j = oc * CA + v * 16 + iota
        tok_v[v // 2][pl.ds((v % 2) * 16, 16)] = j >> 1
    pltpu.sync_copy(pstage_v, pos_hbm.at[pl.ds(oc * CA, CA)])

    @pl.when(oc == 0)
    def _():
        pens = [_lane(pend, e) for e in range(_E)]
        for u in range(2):
            bs = (iota + u * 16) * _BLK
            cntv = jnp.zeros((16,), jnp.int32)
            for e in range(_E):
                cntv = cntv + (bs >= pens[e]).astype(jnp.int32)
            blk_v[pl.ds(u * 16, 16)] = jnp.minimum(cntv, _E - 1)
        pltpu.sync_copy(blk_v, blk_hbm)
        nblk = jnp.sum(padded) >> 8
        nblk_v[...] = jnp.where(iota == 0, nblk, 0)
        pltpu.sync_copy(nblk_v, nblk_hbm)

    bufs = (rows_a, rows_b)
    gh = [None] * 4
    sh = [None] * 4
    gh[0] = pltpu.async_copy(x_hbm.at[tok_v[0]], bufs[0], gsem)
    for cc in range(4):
        gh[cc].wait()
        if cc + 1 < 4:
            if cc >= 1:
                sh[cc - 1].wait()
            gh[cc + 1] = pltpu.async_copy(x_hbm.at[tok_v[cc + 1]],
                                          bufs[(cc + 1) % 2], gsem)
        sh[cc] = pltpu.async_copy(bufs[cc % 2], xg_hbm.at[pidx_v[cc]], ssem)
    sh[2].wait()
    sh[3].wait()


def _combine_body(TT, D, y_hbm, ysh_hbm, pos_hbm, wa_hbm, out_hbm,
                  pos_v, wa_v, rows_v, ysh_v, out_v, sem):
    c = lax.axis_index("c")
    s = lax.axis_index("s")
    g = 2 * s + c
    TPW = TT // (_NC * _NS)
    pltpu.sync_copy(wa_hbm.at[pl.ds(g * 2 * TPW, 2 * TPW)], wa_v)
    NM = D // 16
    for cc in range(TPW // 16):
        tb = g * TPW + cc * 16
        pltpu.sync_copy(pos_hbm.at[pl.ds(tb * 2, 32)], pos_v)
        cp = pltpu.async_copy(y_hbm.at[pos_v], rows_v, sem)
        pltpu.sync_copy(ysh_hbm.at[pl.ds(tb, 16)], ysh_v)
        cp.wait()
        wv_a = wa_v[pl.ds(cc * 32, 16)]
        wv_b = wa_v[pl.ds(cc * 32 + 16, 16)]
        for t in range(16):
            wv = wv_a if t < 8 else wv_b
            w0 = wv[(2 * t) % 16]
            w1 = wv[(2 * t + 1) % 16]

            def mbody(m, _, t=t, w0=w0, w1=w1):
                sl = pl.ds(m * 16, 16)
                out_v[t, sl] = (w0 * rows_v[2 * t, sl]
                                + w1 * rows_v[2 * t + 1, sl]
                                + ysh_v[t, sl])
                return 0

            lax.fori_loop(0, NM, mbody, 0)
        pltpu.sync_copy(out_v, out_hbm.at[pl.ds(tb, 16)])


def kernel(x, Wg, gate_w, up_w, down_w, sh_gate, sh_up, sh_down):
    B, T, D = x.shape
    E, H, _ = gate_w.shape
    TT = B * T
    NA = 2 * TT
    NBLK = (NA + E * (_BLK - 1)) // _BLK + 1
    P = NBLK * _BLK
    NH = H // _HT
    xf = x.reshape(TT, D)

    ea, wa, aux = pl.pallas_call(
        _router_body,
        out_shape=(
            jax.ShapeDtypeStruct((TT, 2), jnp.int32),
            jax.ShapeDtypeStruct((TT, 2), jnp.float32),
            jax.ShapeDtypeStruct((1, 1), jnp.float32),
        ),
        out_specs=(
            pl.BlockSpec(memory_space=pltpu.VMEM),
            pl.BlockSpec(memory_space=pltpu.VMEM),
            pl.BlockSpec(memory_space=pltpu.SMEM),
        ),
    )(xf, Wg)
    eaf = ea.reshape(NA)
    waf = wa.reshape(NA)

    mesh = plsc.VectorSubcoreMesh(core_axis_name="c", subcore_axis_name="s",
                                  num_cores=_NC, num_subcores=_NS)

    d1 = pl.kernel(
        functools.partial(_dispatch_body, NA, P),
        compiler_params=pltpu.CompilerParams(needs_layout_passes=False),
        out_type=(
            jax.ShapeDtypeStruct((NA,), jnp.int32),
            jax.ShapeDtypeStruct((P, D), jnp.float32),
            jax.ShapeDtypeStruct((32,), jnp.int32),
            jax.ShapeDtypeStruct((16,), jnp.int32),
        ),
        mesh=mesh,
        scratch_types=[
            pltpu.VMEM((256,), jnp.int32),
            pltpu.VMEM((32,), jnp.int32),
            pltpu.VMEM((512,), jnp.int32),
            pltpu.VMEM((128,), jnp.int32),
            tuple(pltpu.VMEM((32,), jnp.int32) for _ in range(4)),
            tuple(pltpu.VMEM((32,), jnp.int32) for _ in range(4)),
            pltpu.VMEM((32,), jnp.int32),
            pltpu.VMEM((16,), jnp.int32),
            pltpu.VMEM((32, D), jnp.float32),
            pltpu.VMEM((32, D), jnp.float32),
            pltpu.VMEM_SHARED((512,), jnp.int32),
            pltpu.SemaphoreType.DMA,
            pltpu.SemaphoreType.DMA,
        ],
    )
    pos, xg, blk_e, nblk = d1(eaf, xf)

    def _bc(b, nb):
        return jnp.maximum(jnp.minimum(b, nb[0] - 1), 0)

    grid_spec = pltpu.PrefetchScalarGridSpec(
        num_scalar_prefetch=2,
        grid=(NBLK, NH),
        in_specs=[
            pl.BlockSpec((_BLK, D), lambda b, h, be, nb: (_bc(b, nb), 0)),
            pl.BlockSpec((1, _HT, D),
                         lambda b, h, be, nb: (be[_bc(b, nb)], h, 0)),
            pl.BlockSpec((1, _HT, D),
                         lambda b, h, be, nb: (be[_bc(b, nb)], h, 0)),
            pl.BlockSpec((1, D, _HT),
                         lambda b, h, be, nb: (be[_bc(b, nb)], 0, h)),
        ],
        out_specs=pl.BlockSpec((_BLK, D), lambda b, h, be, nb: (_bc(b, nb), 0)),
    )
    y = pl.pallas_call(
        _moe_ffn_body,
        grid_spec=grid_spec,
        out_shape=jax.ShapeDtypeStruct((P, D), jnp.float32),
        compiler_params=pltpu.CompilerParams(
            dimension_semantics=("arbitrary", "arbitrary"),
        ),
    )(blk_e, nblk, xg, gate_w, up_w, down_w)

    TB = min(512, TT)
    NTB = TT // TB
    ysh = pl.pallas_call(
        _sh_ffn_body,
        grid=(NTB, NH),
        in_specs=[
            pl.BlockSpec((TB, D), lambda b, h: (b, 0)),
            pl.BlockSpec((_HT, D), lambda b, h: (h, 0)),
            pl.BlockSpec((_HT, D), lambda b, h: (h, 0)),
            pl.BlockSpec((D, _HT), lambda b, h: (0, h)),
        ],
        out_specs=pl.BlockSpec((TB, D), lambda b, h: (b, 0)),
        out_shape=jax.ShapeDtypeStruct((TT, D), jnp.float32),
        compiler_params=pltpu.CompilerParams(
            dimension_semantics=("arbitrary", "arbitrary"),
        ),
    )(xf, sh_gate, sh_up, sh_down)

    d3 = pl.kernel(
        functools.partial(_combine_body, TT, D),
        out_type=jax.ShapeDtypeStruct((TT, D), jnp.float32),
        mesh=mesh,
        scratch_types=[
            pltpu.VMEM((32,), jnp.int32),
            pltpu.VMEM((128,), jnp.float32),
            pltpu.VMEM((32, D), jnp.float32),
            pltpu.VMEM((16, D), jnp.float32),
            pltpu.VMEM((16, D), jnp.float32),
            pltpu.SemaphoreType.DMA,
        ],
    )
    out = d3(y, ysh, pos, waf)

    return out.reshape(B, T, D), aux.reshape(())


# speedup vs baseline: 1.8573x; 1.2604x over previous
import functools

import jax
import jax.numpy as jnp
from jax import lax
from jax.experimental import pallas as pl
from jax.experimental.pallas import tpu as pltpu
from jax.experimental.pallas import tpu_sc as plsc

_COEFF = 0.01
_BIG_I = 2**30
_BLK = 512
_BLK_SHIFT = 9
_HT = 256
_NC = 2
_NS = 16
_E = 8


def _router_body(x_ref, wg_ref, ea_ref, wa_ref, aux_ref):
    xf = x_ref[...]
    wg = wg_ref[...]
    T, _ = xf.shape
    E = wg.shape[0]
    logits = jnp.dot(xf, wg.T, preferred_element_type=jnp.float32)
    iota = lax.broadcasted_iota(jnp.int32, logits.shape, 1)
    m1 = jnp.max(logits, axis=-1, keepdims=True)
    a1 = jnp.min(jnp.where(logits == m1, iota, _BIG_I), axis=-1, keepdims=True)
    oh1 = (iota == a1)
    rest = jnp.where(oh1, -jnp.inf, logits)
    m2 = jnp.max(rest, axis=-1, keepdims=True)
    a2 = jnp.min(jnp.where(rest == m2, iota, _BIG_I), axis=-1, keepdims=True)
    e2 = jnp.exp(m2 - m1)
    denom = 1.0 + e2
    ea_ref[...] = jnp.concatenate([a1, a2], axis=1)
    wa_ref[...] = jnp.concatenate([1.0 / denom, e2 / denom], axis=1)
    gates = jnp.exp(logits - m1)
    gates = gates / jnp.sum(gates, axis=-1, keepdims=True)
    f = jnp.mean(oh1.astype(jnp.float32), axis=0)
    P = jnp.mean(gates, axis=0)
    aux_ref[0, 0] = _COEFF * E * jnp.sum(f * P)


def _gelu(g):
    return 0.5 * g * (1.0 + lax.erf(g * (2.0 ** -0.5)))


def _moe_ffn_body(blk_e_ref, nblk_ref, xg_ref, gw_ref, uw_ref, dw_ref, y_ref):
    b = pl.program_id(0)
    h = pl.program_id(1)

    @pl.when(b < nblk_ref[0])
    def _():
        xblk = xg_ref[...]
        g = jnp.dot(xblk, gw_ref[0].T, preferred_element_type=jnp.float32,
                    precision=lax.Precision.DEFAULT)
        u = jnp.dot(xblk, uw_ref[0].T, preferred_element_type=jnp.float32,
                    precision=lax.Precision.DEFAULT)
        hpart = _gelu(g) * u
        yp = lax.dot_general(hpart, dw_ref[0], (((1,), (1,)), ((), ())),
                             preferred_element_type=jnp.float32,
                             precision=lax.Precision.DEFAULT)

        @pl.when(h == 0)
        def _():
            y_ref[...] = jnp.zeros_like(y_ref)

        y_ref[...] += yp


def _sh_ffn_body(x_ref, gw_ref, uw_ref, dw_ref, y_ref):
    h = pl.program_id(1)
    xblk = x_ref[...]
    g = jnp.dot(xblk, gw_ref[...].T, preferred_element_type=jnp.float32,
                precision=lax.Precision.DEFAULT)
    u = jnp.dot(xblk, uw_ref[...].T, preferred_element_type=jnp.float32,
                precision=lax.Precision.DEFAULT)
    hpart = _gelu(g) * u
    yp = lax.dot_general(hpart, dw_ref[...], (((1,), (1,)), ((), ())),
                         preferred_element_type=jnp.float32,
                         precision=lax.Precision.DEFAULT)

    @pl.when(h == 0)
    def _():
        y_ref[...] = jnp.zeros_like(y_ref)

    y_ref[...] += yp


def _iota16():
    return lax.broadcasted_iota(jnp.int32, (16,), 0)


def _lane(vec, e):
    return jnp.sum(jnp.where(_iota16() == e, vec, 0))


def _dispatch_body(NA, P, ea_hbm, x_hbm, pos_hbm, xg_hbm, blk_hbm, nblk_hbm,
                   ea_v, loc_v, cnt_v, pstage_v, pidx_v, tok_v, blk_v, nblk_v,
                   rows_a, rows_b, shared_cnt, gsem, ssem):
    c = lax.axis_index("c")
    s = lax.axis_index("s")
    oc = 2 * s + c
    iota = _iota16()
    CA = NA // (_NC * _NS)
    NV = CA // 16

    pltpu.sync_copy(ea_hbm.at[pl.ds(s * 2 * CA, 2 * CA)], ea_v)
    for cl in range(2):
        cnt = [jnp.int32(0)] * _E
        for v in range(NV):
            vec = ea_v[pl.ds(cl * CA + v * 16, 16)]
            for e in range(_E):
                cnt[e] = cnt[e] + jnp.sum((vec == e).astype(jnp.int32))
        row = jnp.zeros((16,), jnp.int32)
        for e in range(_E):
            row = jnp.where(iota == e, cnt[e], row)
        loc_v[pl.ds(cl * 16, 16)] = row
    pltpu.sync_copy(loc_v, shared_cnt.at[pl.ds(s * 32, 32)])
    plsc.subcore_barrier()

    pltpu.sync_copy(shared_cnt, cnt_v)
    totals = jnp.zeros((16,), jnp.int32)
    prior = jnp.zeros((16,), jnp.int32)
    for w in range(_NC * _NS):
        roww = cnt_v[pl.ds(w * 16, 16)]
        totals = totals + roww
        prior = prior + roww * jnp.where(jnp.int32(w) < oc, 1, 0)
    padded = ((totals + (_BLK - 1)) >> _BLK_SHIFT) << _BLK_SHIFT
    inc_pad = plsc.cumsum(padded)
    pstart = inc_pad - padded
    pend = inc_pad
    base_vec = pstart + prior

    base = [_lane(base_vec, e) for e in range(_E)]
    for v in range(NV):
        va = ea_v[pl.ds(v * 16, 16)]
        vb = ea_v[pl.ds(CA + v * 16, 16)]
        vec = jnp.where(c == 1, vb, va)
        poss = jnp.zeros((16,), jnp.int32)
        for e in range(_E):
            m = vec == e
            mi = m.astype(jnp.int32)
            incl = plsc.cumsum(mi)
            poss = jnp.where(m, base[e] + incl - 1, poss)
            base[e] = base[e] + jnp.sum(mi)
        pstage_v[pl.ds(v * 16, 16)] = poss
        pidx_v[v // 2][pl.ds((v % 2) * 16, 16)] = poss
        j = oc * CA + v * 16 + iota
        tok_v[v // 2][pl.ds((v % 2) * 16, 16)] = j >> 1
    pltpu.sync_copy(pstage_v, pos_hbm.at[pl.ds(oc * CA, CA)])

    @pl.when(oc == 0)
    def _():
        pens = [_lane(pend, e) for e in range(_E)]
        for u in range(2):
            bs = (iota + u * 16) * _BLK
            cntv = jnp.zeros((16,), jnp.int32)
            for e in range(_E):
                cntv = cntv + (bs >= pens[e]).astype(jnp.int32)
            blk_v[pl.ds(u * 16, 16)] = jnp.minimum(cntv, _E - 1)
        pltpu.sync_copy(blk_v, blk_hbm)
        nblk = jnp.sum(padded) >> _BLK_SHIFT
        nblk_v[...] = jnp.where(iota == 0, nblk, 0)
        pltpu.sync_copy(nblk_v, nblk_hbm)

    bufs = (rows_a, rows_b)
    gh = [None] * 4
    sh = [None] * 4
    gh[0] = pltpu.async_copy(x_hbm.at[tok_v[0]], bufs[0], gsem)
    for cc in range(4):
        gh[cc].wait()
        if cc + 1 < 4:
            if cc >= 1:
                sh[cc - 1].wait()
            gh[cc + 1] = pltpu.async_copy(x_hbm.at[tok_v[cc + 1]],
                                          bufs[(cc + 1) % 2], gsem)
        sh[cc] = pltpu.async_copy(bufs[cc % 2], xg_hbm.at[pidx_v[cc]], ssem)
    sh[2].wait()
    sh[3].wait()


def _combine_body(TT, D, y_hbm, ysh_hbm, pos_hbm, wa_hbm, out_hbm,
                  pos_v, wa_v, rows_v, ysh_v, out_v, sem):
    c = lax.axis_index("c")
    s = lax.axis_index("s")
    g = 2 * s + c
    TPW = TT // (_NC * _NS)
    pltpu.sync_copy(wa_hbm.at[pl.ds(g * 2 * TPW, 2 * TPW)], wa_v)
    NM = D // 16
    for cc in range(TPW // 16):
        tb = g * TPW + cc * 16
        pltpu.sync_copy(pos_hbm.at[pl.ds(tb * 2, 32)], pos_v)
        cp = pltpu.async_copy(y_hbm.at[pos_v], rows_v, sem)
        pltpu.sync_copy(ysh_hbm.at[pl.ds(tb, 16)], ysh_v)
        cp.wait()
        wv_a = wa_v[pl.ds(cc * 32, 16)]
        wv_b = wa_v[pl.ds(cc * 32 + 16, 16)]
        for t in range(16):
            wv = wv_a if t < 8 else wv_b
            w0 = wv[(2 * t) % 16]
            w1 = wv[(2 * t + 1) % 16]

            def mbody(m, _, t=t, w0=w0, w1=w1):
                sl = pl.ds(m * 16, 16)
                out_v[t, sl] = (w0 * rows_v[2 * t, sl]
                                + w1 * rows_v[2 * t + 1, sl]
                                + ysh_v[t, sl])
                return 0

            lax.fori_loop(0, NM, mbody, 0)
        pltpu.sync_copy(out_v, out_hbm.at[pl.ds(tb, 16)])


def kernel(x, Wg, gate_w, up_w, down_w, sh_gate, sh_up, sh_down):
    B, T, D = x.shape
    E, H, _ = gate_w.shape
    TT = B * T
    NA = 2 * TT
    NBLK = (NA + E * (_BLK - 1)) // _BLK + 1
    P = NBLK * _BLK
    NH = H // _HT
    xf = x.reshape(TT, D)

    ea, wa, aux = pl.pallas_call(
        _router_body,
        out_shape=(
            jax.ShapeDtypeStruct((TT, 2), jnp.int32),
            jax.ShapeDtypeStruct((TT, 2), jnp.float32),
            jax.ShapeDtypeStruct((1, 1), jnp.float32),
        ),
        out_specs=(
            pl.BlockSpec(memory_space=pltpu.VMEM),
            pl.BlockSpec(memory_space=pltpu.VMEM),
            pl.BlockSpec(memory_space=pltpu.SMEM),
        ),
    )(xf, Wg)
    eaf = ea.reshape(NA)
    waf = wa.reshape(NA)

    mesh = plsc.VectorSubcoreMesh(core_axis_name="c", subcore_axis_name="s",
                                  num_cores=_NC, num_subcores=_NS)

    d1 = pl.kernel(
        functools.partial(_dispatch_body, NA, P),
        compiler_params=pltpu.CompilerParams(needs_layout_passes=False),
        out_type=(
            jax.ShapeDtypeStruct((NA,), jnp.int32),
            jax.ShapeDtypeStruct((P, D), jnp.float32),
            jax.ShapeDtypeStruct((32,), jnp.int32),
            jax.ShapeDtypeStruct((16,), jnp.int32),
        ),
        mesh=mesh,
        scratch_types=[
            pltpu.VMEM((256,), jnp.int32),
            pltpu.VMEM((32,), jnp.int32),
            pltpu.VMEM((512,), jnp.int32),
            pltpu.VMEM((128,), jnp.int32),
            tuple(pltpu.VMEM((32,), jnp.int32) for _ in range(4)),
            tuple(pltpu.VMEM((32,), jnp.int32) for _ in range(4)),
            pltpu.VMEM((32,), jnp.int32),
            pltpu.VMEM((16,), jnp.int32),
            pltpu.VMEM((32, D), jnp.float32),
            pltpu.VMEM((32, D), jnp.float32),
            pltpu.VMEM_SHARED((512,), jnp.int32),
            pltpu.SemaphoreType.DMA,
            pltpu.SemaphoreType.DMA,
        ],
    )
    pos, xg, blk_e, nblk = d1(eaf, xf)

    def _bc(b, nb):
        return jnp.maximum(jnp.minimum(b, nb[0] - 1), 0)

    grid_spec = pltpu.PrefetchScalarGridSpec(
        num_scalar_prefetch=2,
        grid=(NBLK, NH),
        in_specs=[
            pl.BlockSpec((_BLK, D), lambda b, h, be, nb: (_bc(b, nb), 0)),
            pl.BlockSpec((1, _HT, D),
                         lambda b, h, be, nb: (be[_bc(b, nb)], h, 0)),
            pl.BlockSpec((1, _HT, D),
                         lambda b, h, be, nb: (be[_bc(b, nb)], h, 0)),
            pl.BlockSpec((1, D, _HT),
                         lambda b, h, be, nb: (be[_bc(b, nb)], 0, h)),
        ],
        out_specs=pl.BlockSpec((_BLK, D), lambda b, h, be, nb: (_bc(b, nb), 0)),
    )
    y = pl.pallas_call(
        _moe_ffn_body,
        grid_spec=grid_spec,
        out_shape=jax.ShapeDtypeStruct((P, D), jnp.float32),
        compiler_params=pltpu.CompilerParams(
            dimension_semantics=("arbitrary", "arbitrary"),
        ),
    )(blk_e, nblk, xg, gate_w, up_w, down_w)

    TB = min(512, TT)
    NTB = TT // TB
    ysh = pl.pallas_call(
        _sh_ffn_body,
        grid=(NTB, NH),
        in_specs=[
            pl.BlockSpec((TB, D), lambda b, h: (b, 0)),
            pl.BlockSpec((_HT, D), lambda b, h: (h, 0)),
            pl.BlockSpec((_HT, D), lambda b, h: (h, 0)),
            pl.BlockSpec((D, _HT), lambda b, h: (0, h)),
        ],
        out_specs=pl.BlockSpec((TB, D), lambda b, h: (b, 0)),
        out_shape=jax.ShapeDtypeStruct((TT, D), jnp.float32),
        compiler_params=pltpu.CompilerParams(
            dimension_semantics=("arbitrary", "arbitrary"),
        ),
    )(xf, sh_gate, sh_up, sh_down)

    d3 = pl.kernel(
        functools.partial(_combine_body, TT, D),
        out_type=jax.ShapeDtypeStruct((TT, D), jnp.float32),
        mesh=mesh,
        scratch_types=[
            pltpu.VMEM((32,), jnp.int32),
            pltpu.VMEM((128,), jnp.float32),
            pltpu.VMEM((32, D), jnp.float32),
            pltpu.VMEM((16, D), jnp.float32),
            pltpu.VMEM((16, D), jnp.float32),
            pltpu.SemaphoreType.DMA,
        ],
    )
    out = d3(y, ysh, pos, waf)

    return out.reshape(B, T, D), aux.reshape(())
